# Initial kernel scaffold; baseline (speedup 1.0000x reference)
import jax
import jax.numpy as jnp
from jax.experimental import pallas as pl


def kernel(X, ms, sel):
    raise NotImplementedError("write your pallas kernel here")


# speedup vs baseline: 2.1702x; 2.1702x over previous
import jax
import jax.numpy as jnp
from jax.experimental import pallas as pl

_BB = 512


def _dsq_kernel(sel_ref, ms_ref, out_ref):
    k = ms_ref.shape[1] - 1
    qs = ms_ref[:, :k] + ms_ref[:, k:k + 1]
    logqs = jnp.log(qs)
    fire = 1.0 - sel_ref[...].astype(jnp.float32)
    acc = jnp.dot(fire, logqs, preferred_element_type=jnp.float32)
    res = jnp.exp(acc)
    res = jnp.where(res <= 1e-16, res + 1e-16, res)
    out_ref[...] = res / jnp.sum(res, axis=1, keepdims=True)


def kernel(X, ms, sel):
    b, n = sel.shape
    k = ms.shape[1] - 1
    grid = (b // _BB,)
    return pl.pallas_call(
        _dsq_kernel,
        grid=grid,
        in_specs=[
            pl.BlockSpec((_BB, n), lambda i: (i, 0)),
            pl.BlockSpec((n, k + 1), lambda i: (0, 0)),
        ],
        out_specs=pl.BlockSpec((_BB, k), lambda i: (i, 0)),
        out_shape=jax.ShapeDtypeStruct((b, k), jnp.float32),
    )(sel, ms)


# speedup vs baseline: 2.7031x; 1.2456x over previous
import jax
import jax.numpy as jnp
from jax.experimental import pallas as pl

_BB = 4096


def _dsq_kernel(sel_ref, ms_ref, out_ref):
    k = ms_ref.shape[1] - 1
    qs = ms_ref[:, :k] + ms_ref[:, k:k + 1]
    logqs = jnp.log(qs)
    fire = 1.0 - sel_ref[...].astype(jnp.float32)
    acc = jnp.dot(fire, logqs, preferred_element_type=jnp.float32)
    res = jnp.exp(acc)
    res = jnp.where(res <= 1e-16, res + 1e-16, res)
    out_ref[...] = res / jnp.sum(res, axis=1, keepdims=True)


def kernel(X, ms, sel):
    b, n = sel.shape
    k = ms.shape[1] - 1
    grid = (b // _BB,)
    return pl.pallas_call(
        _dsq_kernel,
        grid=grid,
        in_specs=[
            pl.BlockSpec((_BB, n), lambda i: (i, 0)),
            pl.BlockSpec((n, k + 1), lambda i: (0, 0)),
        ],
        out_specs=pl.BlockSpec((_BB, k), lambda i: (i, 0)),
        out_shape=jax.ShapeDtypeStruct((b, k), jnp.float32),
    )(sel, ms)


# speedup vs baseline: 2.8086x; 1.0390x over previous
import jax
import jax.numpy as jnp
from jax.experimental import pallas as pl

_BB = 2048


def _dsq_kernel(sel_ref, ms_ref, out_ref):
    k = ms_ref.shape[1] - 1
    qs = ms_ref[:, :k] + ms_ref[:, k:k + 1]
    logqs = jnp.log(qs)
    fire = 1.0 - sel_ref[...].astype(jnp.float32)
    acc = jnp.dot(fire, logqs, preferred_element_type=jnp.float32)
    res = jnp.exp(acc)
    res = jnp.where(res <= 1e-16, res + 1e-16, res)
    out_ref[...] = res / jnp.sum(res, axis=1, keepdims=True)


def kernel(X, ms, sel):
    b, n = sel.shape
    k = ms.shape[1] - 1
    grid = (b // _BB,)
    return pl.pallas_call(
        _dsq_kernel,
        grid=grid,
        in_specs=[
            pl.BlockSpec((_BB, n), lambda i: (i, 0)),
            pl.BlockSpec((n, k + 1), lambda i: (0, 0)),
        ],
        out_specs=pl.BlockSpec((_BB, k), lambda i: (i, 0)),
        out_shape=jax.ShapeDtypeStruct((b, k), jnp.float32),
    )(sel, ms)
